# baseline (speedup 1.0000x reference)
import functools

import jax
import jax.numpy as jnp
from jax import lax
from jax.experimental import pallas as pl
from jax.experimental.pallas import tpu as pltpu
from jax.experimental.pallas import tpu_sc as plsc

_B, _L, _T, _H = 16, 2048, 4096, 256
_NF = _B * _L
_CH = 128
_NW = 32
_BT = 256
_NTB = _T // _BT
_LB = _L // _BT
_D6 = 6 * _H
_DH = 4 * _H
_TTR = 136
_GTR = 16


def _sc_gather4(tok_tbl, ids4):
    n4 = 4 * _NF // _NW
    ch = 128
    nb = 4
    nc = n4 // ch
    hw = _H // 2
    mesh = plsc.VectorSubcoreMesh(core_axis_name="c", subcore_axis_name="s")
    out_t = jax.ShapeDtypeStruct((4 * _NF, hw), jnp.int32)

    @functools.partial(
        pl.kernel, mesh=mesh, out_type=out_t,
        scratch_types=[pltpu.VMEM((n4,), jnp.int32),
                       pltpu.VMEM((nb, ch, hw), jnp.int32),
                       pltpu.SemaphoreType.DMA((nb,)),
                       pltpu.SemaphoreType.DMA((nb,))])
    def k(tt, i4, o4, i4_v, buf, sg, ss):
        wid = lax.axis_index("s") * 2 + lax.axis_index("c")
        base = wid * n4

        def g_start(c, par):
            pltpu.async_copy(tt.at[i4_v.at[pl.ds(c * ch, ch)]],
                             buf.at[par], sg.at[par])

        def g_wait(par):
            pltpu.make_async_copy(tt.at[i4_v.at[pl.ds(0, ch)]],
                                  buf.at[par], sg.at[par]).wait()

        def s_start(c, par):
            pltpu.async_copy(buf.at[par],
                             o4.at[pl.ds(base + c * ch, ch)],
                             ss.at[par])

        def s_wait(par):
            pltpu.make_async_copy(buf.at[0],
                                  o4.at[pl.ds(base, ch)],
                                  ss.at[par]).wait()

        pltpu.sync_copy(i4.at[pl.ds(base, n4)], i4_v)
        g_start(0, 0)
        g_start(1, 1)
        g_start(2, 2)
        g_wait(0)
        s_start(0, 0)
        g_start(3, 3)
        for k in (1, 2, 3):
            g_wait(k % nb)
            s_wait((k - 1) % nb)
            s_start(k, k % nb)
            g_start(k + 3, (k + 3) % nb)

        def body(p, carry):
            for par in (0, 1, 2, 3):
                k = 4 * p + par
                g_wait(par)
                s_wait((par - 1) % nb)
                s_start(k, par)
                g_start(k + 3, (par + 3) % nb)
            return carry
        lax.fori_loop(1, (nc - 4) // nb, body, 0)
        k = nc - 4
        g_wait(k % nb)
        s_wait((k - 1) % nb)
        s_start(k, k % nb)
        g_start(k + 3, (k + 3) % nb)
        for k in (nc - 3, nc - 2, nc - 1):
            g_wait(k % nb)
            s_wait((k - 1) % nb)
            s_start(k, k % nb)
        s_wait((nc - 1) % nb)

    return k(tok_tbl, ids4)


def _mlp_body(n_ref, x0, tid_ref, gid_ref, tt_ref, gt_ref, g_ref, be_ref,
              w1_ref, b1_ref, w2_ref, b2_ref, o_ref):
    b = pl.program_id(0)
    i = pl.program_id(1)
    nb = n_ref[b]

    @pl.when(i * _BT < nb)
    def _compute():
        tn = (((0,), (0,)), ((), ()))
        iot = lax.broadcasted_iota(jnp.int32, (_TTR, _BT), 0)
        oht = (iot == jnp.broadcast_to(tid_ref[0], (_TTR, _BT))).astype(
            jnp.bfloat16)
        th = lax.dot_general(oht, tt_ref[...], dimension_numbers=tn,
                             preferred_element_type=jnp.float32)
        iog = lax.broadcasted_iota(jnp.int32, (_GTR, _BT), 0)
        ohg = (iog == jnp.broadcast_to(gid_ref[0], (_GTR, _BT))).astype(
            jnp.bfloat16)
        gh = lax.dot_general(ohg, gt_ref[...], dimension_numbers=tn,
                             preferred_element_type=jnp.float32)
        x = jnp.concatenate([x0[0].astype(jnp.float32), th, gh],
                            axis=-1)
        mu = jnp.mean(x, axis=1, keepdims=True)
        var = jnp.mean(x * x, axis=1, keepdims=True) - mu * mu
        xn = (x - mu) * lax.rsqrt(var + 1e-5)
        xn = xn * g_ref[0] + be_ref[0]
        h = jnp.dot(xn.astype(jnp.bfloat16), w1_ref[...],
                    preferred_element_type=jnp.float32) + b1_ref[0]
        a = h * jax.nn.sigmoid(h)
        o = jnp.dot(a.astype(jnp.bfloat16), w2_ref[...],
                    preferred_element_type=jnp.float32) + b2_ref[0]
        o_ref[0] = o.astype(jnp.bfloat16)

    @pl.when(i * _BT >= nb)
    def _zero():
        o_ref[...] = jnp.zeros_like(o_ref)


def _mlp(xt, tid_r, gid_r, tt_pad, gt_pad, n_arr, gamma, beta,
         w1t, b1, w2t, b2):
    grid_spec = pltpu.PrefetchScalarGridSpec(
        num_scalar_prefetch=1,
        grid=(_B, _LB),
        in_specs=[
            pl.BlockSpec((1, _BT, 4 * _H), lambda b, i, n: (b, i, 0)),
            pl.BlockSpec((1, 1, _BT), lambda b, i, n: (b * _LB + i, 0, 0)),
            pl.BlockSpec((1, 1, _BT), lambda b, i, n: (b * _LB + i, 0, 0)),
            pl.BlockSpec((_TTR, _H), lambda b, i, n: (0, 0)),
            pl.BlockSpec((_GTR, _H), lambda b, i, n: (0, 0)),
            pl.BlockSpec((1, 1, _D6), lambda b, i, n: (0, 0, 0)),
            pl.BlockSpec((1, 1, _D6), lambda b, i, n: (0, 0, 0)),
            pl.BlockSpec((_D6, _DH), lambda b, i, n: (0, 0)),
            pl.BlockSpec((1, 1, _DH), lambda b, i, n: (0, 0, 0)),
            pl.BlockSpec((_DH, _H), lambda b, i, n: (0, 0)),
            pl.BlockSpec((1, 1, _H), lambda b, i, n: (0, 0, 0)),
        ],
        out_specs=pl.BlockSpec((1, _BT, _H), lambda b, i, n: (b, i, 0)),
    )
    return pl.pallas_call(
        _mlp_body, grid_spec=grid_spec,
        out_shape=jax.ShapeDtypeStruct((_B, _L, _H), jnp.bfloat16),
    )(n_arr, xt, tid_r, gid_r, tt_pad, gt_pad, gamma, beta, w1t, b1, w2t, b2)


def _merge_body(j_ref, j2_ref, evA, evB, p_ref, s_ref, pos_ref, sep_ref,
                o_ref):
    b = pl.program_id(0)
    t = pl.program_id(1)
    j = j_ref[b * _NTB + t]
    W = 4 * _BT
    io0 = lax.broadcasted_iota(jnp.int32, (W, _BT), 0)
    io1 = lax.broadcasted_iota(jnp.int32, (W, _BT), 1)
    pid_b = jnp.broadcast_to(p_ref[0], (W, _BT))
    sl_b = jnp.broadcast_to(s_ref[0], (W, _BT))
    local = pid_b - j * _BT
    oh_ev = (io0 == local) & (sl_b == 1)
    oh_sep = (io0 == 2 * _BT) & (sl_b == 2)
    oh_pos = (io0 - 3 * _BT == io1) & (sl_b != 0)
    ohT = (oh_ev | oh_sep | oh_pos).astype(jnp.bfloat16)
    win = jnp.concatenate([evA[0], evB[0], sep_ref[0], pos_ref[...]],
                          axis=0)
    o_ref[0] = lax.dot_general(
        ohT, win, dimension_numbers=(((0,), (0,)), ((), ())),
        preferred_element_type=jnp.float32)


def _merge(ev, pidx3, sel3, j_arr, j2_arr, pos_tab, sep3):
    grid_spec = pltpu.PrefetchScalarGridSpec(
        num_scalar_prefetch=2,
        grid=(_B, _NTB),
        in_specs=[
            pl.BlockSpec((1, _BT, _H),
                         lambda b, t, j, j2: (b, j[b * _NTB + t], 0)),
            pl.BlockSpec((1, _BT, _H),
                         lambda b, t, j, j2: (b, j2[b * _NTB + t], 0)),
            pl.BlockSpec((1, 1, _BT),
                         lambda b, t, j, j2: (b * _NTB + t, 0, 0)),
            pl.BlockSpec((1, 1, _BT),
                         lambda b, t, j, j2: (b * _NTB + t, 0, 0)),
            pl.BlockSpec((_BT, _H), lambda b, t, j, j2: (t, 0)),
            pl.BlockSpec((1, _BT, _H), lambda b, t, j, j2: (0, 0, 0)),
        ],
        out_specs=pl.BlockSpec((1, _BT, _H), lambda b, t, j, j2: (b, t, 0)),
    )
    return pl.pallas_call(
        _merge_body, grid_spec=grid_spec,
        out_shape=jax.ShapeDtypeStruct((_B, _T, _H), jnp.float32),
    )(j_arr, j2_arr, ev, ev, pidx3, sel3, pos_tab, sep3)


def kernel(history_tokens, history_post_tokens, history_author_tokens,
           history_action_tokens, history_time_gap, history_group_ids,
           history_mask, token_table, time_table, group_table, pos_table,
           ln_gamma, ln_beta, W1, b1, W2, b2, sep_token):
    i32 = jnp.int32
    mask = history_mask.astype(bool)
    group = history_group_ids.astype(i32)

    mi = mask.astype(i32)
    pc = jnp.cumsum(mi, axis=1) - 1
    n_arr = jnp.sum(mi, axis=1).astype(i32)
    idx = jnp.arange(_L, dtype=i32)
    bi = jnp.arange(_B, dtype=i32)[:, None]
    packed_l = jnp.zeros((_B, _L), dtype=i32).at[
        bi, jnp.where(mask, pc, _L)].set(
        jnp.broadcast_to(idx[None, :], (_B, _L)), mode='drop')

    big = jnp.int32(1 << 30)
    gv = jnp.arange(9, dtype=i32)
    cnt = jnp.sum((group[:, None, :] == gv[None, :, None]) &
                  mask[:, None, :], axis=2).astype(i32)
    cum = jnp.cumsum(cnt, axis=1)
    nonempty = cnt > 0
    rev = jnp.cumsum(nonempty[:, ::-1].astype(i32), axis=1)[:, ::-1]
    sep_ex = nonempty & ((rev - nonempty.astype(i32)) > 0)
    sep_rank = jnp.cumsum(sep_ex.astype(i32), axis=1) - sep_ex.astype(i32)
    S = jnp.where(sep_ex, cum + sep_rank, big)
    tot = n_arr + jnp.sum(sep_ex, axis=1).astype(i32)

    k_t = jnp.arange(_T, dtype=i32)[None, :] - (_T - tot[:, None])
    nsep_le = jnp.sum((S[:, :, None] <= k_t[:, None, :]), axis=1)
    is_sep = jnp.any(S[:, :, None] == k_t[:, None, :], axis=1)
    sel = jnp.where(k_t >= 0, jnp.where(is_sep, 2, 1), 0)
    pidx = jnp.where(sel == 1, k_t - nsep_le, -1)
    p4 = pidx.reshape(_B, _NTB, _BT)
    w0 = jnp.min(jnp.where(p4 >= 0, p4, big), axis=2)
    j_arr = jnp.clip(jnp.where(w0 >= big, 0, w0 // _BT), 0, _LB - 1)
    j2_arr = jnp.minimum(j_arr + 1, _LB - 1)
    j_arr = j_arr.reshape(-1).astype(i32)
    j2_arr = j2_arr.reshape(-1).astype(i32)

    ids_all = jnp.stack(
        [history_tokens.astype(i32), history_post_tokens.astype(i32),
         history_author_tokens.astype(i32),
         history_action_tokens.astype(i32),
         jnp.clip(history_time_gap, 0, 128).astype(i32), group], axis=1)
    packed_all = jnp.take_along_axis(
        ids_all, jnp.broadcast_to(packed_l[:, None, :], (_B, 6, _L)),
        axis=2)
    ids4 = packed_all[:, :4].transpose(0, 2, 1).reshape(-1)
    tid_r = packed_all[:, 4].reshape(_B * _LB, 1, _BT)
    gid_r = packed_all[:, 5].reshape(_B * _LB, 1, _BT)
    bf16 = jnp.bfloat16
    tt_pad = jnp.zeros((_TTR, _H), bf16).at[:129].set(
        time_table.astype(bf16))
    gt_pad = jnp.zeros((_GTR, _H), bf16).at[:9].set(
        group_table.astype(bf16))

    tok_i32 = lax.bitcast_convert_type(
        token_table.astype(bf16).reshape(-1, _H // 2, 2), i32)
    xt4 = lax.bitcast_convert_type(
        _sc_gather4(tok_i32, ids4), bf16).reshape(_B, _L, 4 * _H)

    gamma = ln_gamma.reshape(1, 1, _D6)
    beta = ln_beta.reshape(1, 1, _D6)
    w1t = W1.T.astype(bf16)
    w2t = W2.T.astype(bf16)
    ev = _mlp(xt4, tid_r, gid_r, tt_pad, gt_pad, n_arr, gamma, beta, w1t,
              b1.reshape(1, 1, _DH), w2t, b2.reshape(1, 1, _H))

    pidx3 = pidx.reshape(_B * _NTB, 1, _BT)
    sel3 = sel.reshape(_B * _NTB, 1, _BT)
    sep_pad = jnp.zeros((1, _BT, _H), jnp.bfloat16).at[0, 0].set(
        sep_token.astype(jnp.bfloat16))
    merged = _merge(ev, pidx3, sel3, j_arr, j2_arr,
                    pos_table.astype(jnp.bfloat16), sep_pad)
    return merged, sel != 0


# speedup vs baseline: 17.8712x; 17.8712x over previous
import functools

import jax
import jax.numpy as jnp
from jax import lax
from jax.experimental import pallas as pl
from jax.experimental.pallas import tpu as pltpu
from jax.experimental.pallas import tpu_sc as plsc

_B, _L, _T, _H = 16, 2048, 4096, 256
_NF = _B * _L
_CH = 128
_NW = 32
_BT = 256
_NTB = _T // _BT
_LB = _L // _BT
_D6 = 6 * _H
_DH = 4 * _H
_TTR = 136
_GTR = 16


def _sc_gather4(tok_tbl, ids4):
    n4 = 4 * _NF // _NW
    ch = 128
    nb = 4
    nc = n4 // ch
    mesh = plsc.VectorSubcoreMesh(core_axis_name="c", subcore_axis_name="s")
    hw = _H // 2
    out_t = jax.ShapeDtypeStruct((4 * _NF, hw), jnp.int32)

    @functools.partial(
        pl.kernel, mesh=mesh, out_type=out_t,
        scratch_types=[pltpu.VMEM((n4,), jnp.int32),
                       pltpu.VMEM((nb, ch, hw), jnp.int32),
                       pltpu.SemaphoreType.DMA((nb,)),
                       pltpu.SemaphoreType.DMA((nb,))])
    def k(tt, i4, o4, i4_v, buf, sg, ss):
        wid = lax.axis_index("s") * 2 + lax.axis_index("c")
        base = wid * n4

        def g_start(c, par):
            pltpu.async_copy(tt.at[i4_v.at[pl.ds(c * ch, ch)]],
                             buf.at[par], sg.at[par])

        def g_wait(par):
            pltpu.make_async_copy(tt.at[i4_v.at[pl.ds(0, ch)]],
                                  buf.at[par], sg.at[par]).wait()

        def s_start(c, par):
            pltpu.async_copy(buf.at[par],
                             o4.at[pl.ds(base + c * ch, ch)],
                             ss.at[par])

        def s_wait(par):
            pltpu.make_async_copy(buf.at[0],
                                  o4.at[pl.ds(base, ch)],
                                  ss.at[par]).wait()

        pltpu.sync_copy(i4.at[pl.ds(base, n4)], i4_v)
        g_start(0, 0)
        g_start(1, 1)
        g_start(2, 2)
        g_wait(0)
        s_start(0, 0)
        g_start(3, 3)
        for k in (1, 2, 3):
            g_wait(k % nb)
            s_wait((k - 1) % nb)
            s_start(k, k % nb)
            g_start(k + 3, (k + 3) % nb)

        def body(p, carry):
            for par in (0, 1, 2, 3):
                k = 4 * p + par
                g_wait(par)
                s_wait((par - 1) % nb)
                s_start(k, par)
                g_start(k + 3, (par + 3) % nb)
            return carry
        lax.fori_loop(1, (nc - 4) // nb, body, 0)
        k = nc - 4
        g_wait(k % nb)
        s_wait((k - 1) % nb)
        s_start(k, k % nb)
        g_start(k + 3, (k + 3) % nb)
        for k in (nc - 3, nc - 2, nc - 1):
            g_wait(k % nb)
            s_wait((k - 1) % nb)
            s_start(k, k % nb)
        s_wait((nc - 1) % nb)

    return k(tok_tbl, ids4)


def _mlp_body(n_ref, x0, tid_ref, gid_ref, tt_ref, gt_ref, g_ref, be_ref,
              w1_ref, b1_ref, w2_ref, b2_ref, o_ref):
    b = pl.program_id(0)
    i = pl.program_id(1)
    nb = n_ref[b]

    @pl.when(i * _BT < nb)
    def _compute():
        wu = lax.bitcast_convert_type(x0[0], jnp.uint32)
        lo = lax.bitcast_convert_type(wu.astype(jnp.uint16), jnp.bfloat16)
        hi = lax.bitcast_convert_type((wu >> 16).astype(jnp.uint16),
                                      jnp.bfloat16)
        xtok = jnp.concatenate([lo, hi], axis=-1).astype(jnp.float32)
        tn = (((0,), (0,)), ((), ()))
        iot = lax.broadcasted_iota(jnp.int32, (_TTR, _BT), 0)
        oht = (iot == jnp.broadcast_to(tid_ref[0], (_TTR, _BT))).astype(
            jnp.bfloat16)
        th = lax.dot_general(oht, tt_ref[...], dimension_numbers=tn,
                             preferred_element_type=jnp.float32)
        iog = lax.broadcasted_iota(jnp.int32, (_GTR, _BT), 0)
        ohg = (iog == jnp.broadcast_to(gid_ref[0], (_GTR, _BT))).astype(
            jnp.bfloat16)
        gh = lax.dot_general(ohg, gt_ref[...], dimension_numbers=tn,
                             preferred_element_type=jnp.float32)
        x = jnp.concatenate([xtok, th, gh], axis=-1)
        mu = jnp.mean(x, axis=1, keepdims=True)
        var = jnp.mean(x * x, axis=1, keepdims=True) - mu * mu
        xn = (x - mu) * lax.rsqrt(var + 1e-5)
        xn = xn * g_ref[0] + be_ref[0]
        h = jnp.dot(xn.astype(jnp.bfloat16), w1_ref[...],
                    preferred_element_type=jnp.float32) + b1_ref[0]
        a = h * jax.nn.sigmoid(h)
        o = jnp.dot(a.astype(jnp.bfloat16), w2_ref[...],
                    preferred_element_type=jnp.float32) + b2_ref[0]
        o_ref[0] = o.astype(jnp.bfloat16)

    @pl.when(i * _BT >= nb)
    def _zero():
        o_ref[...] = jnp.zeros_like(o_ref)


def _mlp(xt, tid_r, gid_r, tt_pad, gt_pad, n_arr, gamma, beta,
         w1t, b1, w2t, b2):
    grid_spec = pltpu.PrefetchScalarGridSpec(
        num_scalar_prefetch=1,
        grid=(_B, _LB),
        in_specs=[
            pl.BlockSpec((1, _BT, 2 * _H), lambda b, i, n: (b, i, 0)),
            pl.BlockSpec((1, 1, _BT), lambda b, i, n: (b * _LB + i, 0, 0)),
            pl.BlockSpec((1, 1, _BT), lambda b, i, n: (b * _LB + i, 0, 0)),
            pl.BlockSpec((_TTR, _H), lambda b, i, n: (0, 0)),
            pl.BlockSpec((_GTR, _H), lambda b, i, n: (0, 0)),
            pl.BlockSpec((1, 1, _D6), lambda b, i, n: (0, 0, 0)),
            pl.BlockSpec((1, 1, _D6), lambda b, i, n: (0, 0, 0)),
            pl.BlockSpec((_D6, _DH), lambda b, i, n: (0, 0)),
            pl.BlockSpec((1, 1, _DH), lambda b, i, n: (0, 0, 0)),
            pl.BlockSpec((_DH, _H), lambda b, i, n: (0, 0)),
            pl.BlockSpec((1, 1, _H), lambda b, i, n: (0, 0, 0)),
        ],
        out_specs=pl.BlockSpec((1, _BT, _H), lambda b, i, n: (b, i, 0)),
    )
    return pl.pallas_call(
        _mlp_body, grid_spec=grid_spec,
        out_shape=jax.ShapeDtypeStruct((_B, _L, _H), jnp.bfloat16),
    )(n_arr, xt, tid_r, gid_r, tt_pad, gt_pad, gamma, beta, w1t, b1, w2t, b2)


def _merge_body(j_ref, j2_ref, evA, evB, p_ref, s_ref, pos_ref, sep_ref,
                o_ref):
    b = pl.program_id(0)
    t = pl.program_id(1)
    j = j_ref[b * _NTB + t]
    W = 4 * _BT
    io0 = lax.broadcasted_iota(jnp.int32, (W, _BT), 0)
    io1 = lax.broadcasted_iota(jnp.int32, (W, _BT), 1)
    pid_b = jnp.broadcast_to(p_ref[0], (W, _BT))
    sl_b = jnp.broadcast_to(s_ref[0], (W, _BT))
    local = pid_b - j * _BT
    oh_ev = (io0 == local) & (sl_b == 1)
    oh_sep = (io0 == 2 * _BT) & (sl_b == 2)
    oh_pos = (io0 - 3 * _BT == io1) & (sl_b != 0)
    ohT = (oh_ev | oh_sep | oh_pos).astype(jnp.bfloat16)
    win = jnp.concatenate([evA[0], evB[0], sep_ref[0], pos_ref[...]],
                          axis=0)
    o_ref[0] = lax.dot_general(
        ohT, win, dimension_numbers=(((0,), (0,)), ((), ())),
        preferred_element_type=jnp.float32)


def _merge(ev, pidx3, sel3, j_arr, j2_arr, pos_tab, sep3):
    grid_spec = pltpu.PrefetchScalarGridSpec(
        num_scalar_prefetch=2,
        grid=(_B, _NTB),
        in_specs=[
            pl.BlockSpec((1, _BT, _H),
                         lambda b, t, j, j2: (b, j[b * _NTB + t], 0)),
            pl.BlockSpec((1, _BT, _H),
                         lambda b, t, j, j2: (b, j2[b * _NTB + t], 0)),
            pl.BlockSpec((1, 1, _BT),
                         lambda b, t, j, j2: (b * _NTB + t, 0, 0)),
            pl.BlockSpec((1, 1, _BT),
                         lambda b, t, j, j2: (b * _NTB + t, 0, 0)),
            pl.BlockSpec((_BT, _H), lambda b, t, j, j2: (t, 0)),
            pl.BlockSpec((1, _BT, _H), lambda b, t, j, j2: (0, 0, 0)),
        ],
        out_specs=pl.BlockSpec((1, _BT, _H), lambda b, t, j, j2: (b, t, 0)),
    )
    return pl.pallas_call(
        _merge_body, grid_spec=grid_spec,
        out_shape=jax.ShapeDtypeStruct((_B, _T, _H), jnp.float32),
    )(j_arr, j2_arr, ev, ev, pidx3, sel3, pos_tab, sep3)


def kernel(history_tokens, history_post_tokens, history_author_tokens,
           history_action_tokens, history_time_gap, history_group_ids,
           history_mask, token_table, time_table, group_table, pos_table,
           ln_gamma, ln_beta, W1, b1, W2, b2, sep_token):
    i32 = jnp.int32
    mask = history_mask.astype(bool)
    group = history_group_ids.astype(i32)

    mi = mask.astype(i32)
    pc = jnp.cumsum(mi, axis=1) - 1
    n_arr = jnp.sum(mi, axis=1).astype(i32)
    idx = jnp.arange(_L, dtype=i32)
    bi = jnp.arange(_B, dtype=i32)[:, None]
    packed_l = jnp.zeros((_B, _L), dtype=i32).at[
        bi, jnp.where(mask, pc, _L)].set(
        jnp.broadcast_to(idx[None, :], (_B, _L)), mode='drop')

    big = jnp.int32(1 << 30)
    gv = jnp.arange(9, dtype=i32)
    cnt = jnp.sum((group[:, None, :] == gv[None, :, None]) &
                  mask[:, None, :], axis=2).astype(i32)
    cum = jnp.cumsum(cnt, axis=1)
    nonempty = cnt > 0
    rev = jnp.cumsum(nonempty[:, ::-1].astype(i32), axis=1)[:, ::-1]
    sep_ex = nonempty & ((rev - nonempty.astype(i32)) > 0)
    sep_rank = jnp.cumsum(sep_ex.astype(i32), axis=1) - sep_ex.astype(i32)
    S = jnp.where(sep_ex, cum + sep_rank, big)
    tot = n_arr + jnp.sum(sep_ex, axis=1).astype(i32)

    k_t = jnp.arange(_T, dtype=i32)[None, :] - (_T - tot[:, None])
    nsep_le = jnp.sum((S[:, :, None] <= k_t[:, None, :]), axis=1)
    is_sep = jnp.any(S[:, :, None] == k_t[:, None, :], axis=1)
    sel = jnp.where(k_t >= 0, jnp.where(is_sep, 2, 1), 0)
    pidx = jnp.where(sel == 1, k_t - nsep_le, -1)
    p4 = pidx.reshape(_B, _NTB, _BT)
    w0 = jnp.min(jnp.where(p4 >= 0, p4, big), axis=2)
    j_arr = jnp.clip(jnp.where(w0 >= big, 0, w0 // _BT), 0, _LB - 1)
    j2_arr = jnp.minimum(j_arr + 1, _LB - 1)
    j_arr = j_arr.reshape(-1).astype(i32)
    j2_arr = j2_arr.reshape(-1).astype(i32)

    ids_all = jnp.stack(
        [history_tokens.astype(i32), history_post_tokens.astype(i32),
         history_author_tokens.astype(i32),
         history_action_tokens.astype(i32),
         jnp.clip(history_time_gap, 0, 128).astype(i32), group], axis=1)
    packed_all = jnp.take_along_axis(
        ids_all, jnp.broadcast_to(packed_l[:, None, :], (_B, 6, _L)),
        axis=2)
    ids4 = packed_all[:, :4].transpose(0, 2, 1).reshape(-1)
    tid_r = packed_all[:, 4].reshape(_B * _LB, 1, _BT)
    gid_r = packed_all[:, 5].reshape(_B * _LB, 1, _BT)
    bf16 = jnp.bfloat16
    tt_pad = jnp.zeros((_TTR, _H), bf16).at[:129].set(
        time_table.astype(bf16))
    gt_pad = jnp.zeros((_GTR, _H), bf16).at[:9].set(
        group_table.astype(bf16))

    u16v = lax.bitcast_convert_type(token_table.astype(bf16), jnp.uint16)
    tok_i32 = lax.bitcast_convert_type(
        u16v[:, :128].astype(jnp.uint32) |
        (u16v[:, 128:].astype(jnp.uint32) << 16), i32)
    xt4 = _sc_gather4(tok_i32, ids4).reshape(_B, _L, 2 * _H)

    def hperm(v):
        v4 = v[:1024].reshape(4, 2, 128, -1)
        return jnp.concatenate(
            [v4[:, 0].reshape(512, -1), v4[:, 1].reshape(512, -1),
             v[1024:].reshape(512, -1)], axis=0)

    gamma = hperm(ln_gamma).reshape(1, 1, _D6)
    beta = hperm(ln_beta).reshape(1, 1, _D6)
    w1t = hperm(W1.T).astype(bf16)
    w2t = W2.T.astype(bf16)
    ev = _mlp(xt4, tid_r, gid_r, tt_pad, gt_pad, n_arr, gamma, beta, w1t,
              b1.reshape(1, 1, _DH), w2t, b2.reshape(1, 1, _H))

    pidx3 = pidx.reshape(_B * _NTB, 1, _BT)
    sel3 = sel.reshape(_B * _NTB, 1, _BT)
    sep_pad = jnp.zeros((1, _BT, _H), jnp.bfloat16).at[0, 0].set(
        sep_token.astype(jnp.bfloat16))
    merged = _merge(ev, pidx3, sel3, j_arr, j2_arr,
                    pos_table.astype(jnp.bfloat16), sep_pad)
    return merged, sel != 0


# speedup vs baseline: 19.6717x; 1.1007x over previous
import functools

import jax
import jax.numpy as jnp
from jax import lax
from jax.experimental import pallas as pl
from jax.experimental.pallas import tpu as pltpu
from jax.experimental.pallas import tpu_sc as plsc

_B, _L, _T, _H = 16, 2048, 4096, 256
_NF = _B * _L
_CH = 128
_NW = 32
_BT = 256
_NTB = _T // _BT
_LB = _L // _BT
_D6 = 6 * _H
_DH = 4 * _H
_TTR = 136
_GTR = 16


def _sc_gather4(tok_tbl, ids4):
    n4 = 4 * _NF // _NW
    ch = 128
    nb = 4
    nc = n4 // ch
    mesh = plsc.VectorSubcoreMesh(core_axis_name="c", subcore_axis_name="s")
    hw = _H // 2
    out_t = jax.ShapeDtypeStruct((4 * _NF, hw), jnp.int32)

    @functools.partial(
        pl.kernel, mesh=mesh, out_type=out_t,
        scratch_types=[pltpu.VMEM((n4,), jnp.int32),
                       pltpu.VMEM((nb, ch, hw), jnp.int32),
                       pltpu.SemaphoreType.DMA((nb,)),
                       pltpu.SemaphoreType.DMA((nb,))])
    def k(tt, i4, o4, i4_v, buf, sg, ss):
        wid = lax.axis_index("s") * 2 + lax.axis_index("c")
        base = wid * n4

        def g_start(c, par):
            pltpu.async_copy(tt.at[i4_v.at[pl.ds(c * ch, ch)]],
                             buf.at[par], sg.at[par])

        def g_wait(par):
            pltpu.make_async_copy(tt.at[i4_v.at[pl.ds(0, ch)]],
                                  buf.at[par], sg.at[par]).wait()

        def s_start(c, par):
            pltpu.async_copy(buf.at[par],
                             o4.at[pl.ds(base + c * ch, ch)],
                             ss.at[par])

        def s_wait(par):
            pltpu.make_async_copy(buf.at[0],
                                  o4.at[pl.ds(base, ch)],
                                  ss.at[par]).wait()

        pltpu.sync_copy(i4.at[pl.ds(base, n4)], i4_v)
        g_start(0, 0)
        g_start(1, 1)
        g_start(2, 2)
        g_wait(0)
        s_start(0, 0)
        g_start(3, 3)
        for k in (1, 2, 3):
            g_wait(k % nb)
            s_wait((k - 1) % nb)
            s_start(k, k % nb)
            g_start(k + 3, (k + 3) % nb)

        def body(p, carry):
            for par in (0, 1, 2, 3):
                k = 4 * p + par
                g_wait(par)
                s_wait((par - 1) % nb)
                s_start(k, par)
                g_start(k + 3, (par + 3) % nb)
            return carry
        lax.fori_loop(1, (nc - 4) // nb, body, 0)
        k = nc - 4
        g_wait(k % nb)
        s_wait((k - 1) % nb)
        s_start(k, k % nb)
        g_start(k + 3, (k + 3) % nb)
        for k in (nc - 3, nc - 2, nc - 1):
            g_wait(k % nb)
            s_wait((k - 1) % nb)
            s_start(k, k % nb)
        s_wait((nc - 1) % nb)

    return k(tok_tbl, ids4)


def _mlp_body(n_ref, x0, tid_ref, gid_ref, tt_ref, gt_ref, g_ref, be_ref,
              w1_ref, b1_ref, w2_ref, b2_ref, o_ref):
    b = pl.program_id(0)
    i = pl.program_id(1)
    nb = n_ref[b]

    @pl.when(i * _BT < nb)
    def _compute():
        wu = lax.bitcast_convert_type(x0[0], jnp.uint32)
        lo = lax.bitcast_convert_type(wu.astype(jnp.uint16), jnp.bfloat16)
        hi = lax.bitcast_convert_type((wu >> 16).astype(jnp.uint16),
                                      jnp.bfloat16)
        xtok = jnp.concatenate([lo, hi], axis=-1).astype(jnp.float32)
        tn = (((0,), (0,)), ((), ()))
        iot = lax.broadcasted_iota(jnp.int32, (_TTR, _BT), 0)
        oht = (iot == jnp.broadcast_to(tid_ref[0], (_TTR, _BT))).astype(
            jnp.bfloat16)
        th = lax.dot_general(oht, tt_ref[...], dimension_numbers=tn,
                             preferred_element_type=jnp.float32)
        iog = lax.broadcasted_iota(jnp.int32, (_GTR, _BT), 0)
        ohg = (iog == jnp.broadcast_to(gid_ref[0], (_GTR, _BT))).astype(
            jnp.bfloat16)
        gh = lax.dot_general(ohg, gt_ref[...], dimension_numbers=tn,
                             preferred_element_type=jnp.float32)
        x = jnp.concatenate([xtok, th, gh], axis=-1)
        mu = jnp.mean(x, axis=1, keepdims=True)
        var = jnp.mean(x * x, axis=1, keepdims=True) - mu * mu
        xn = (x - mu) * lax.rsqrt(var + 1e-5)
        xn = xn * g_ref[0] + be_ref[0]
        h = jnp.dot(xn.astype(jnp.bfloat16), w1_ref[...],
                    preferred_element_type=jnp.float32) + b1_ref[0]
        a = h * jax.nn.sigmoid(h)
        o = jnp.dot(a.astype(jnp.bfloat16), w2_ref[...],
                    preferred_element_type=jnp.float32) + b2_ref[0]
        o_ref[0] = o.astype(jnp.bfloat16)

    @pl.when(i * _BT >= nb)
    def _zero():
        o_ref[...] = jnp.zeros_like(o_ref)


def _mlp(xt, tid_r, gid_r, tt_pad, gt_pad, n_arr, gamma, beta,
         w1t, b1, w2t, b2):
    grid_spec = pltpu.PrefetchScalarGridSpec(
        num_scalar_prefetch=1,
        grid=(_B, _LB),
        in_specs=[
            pl.BlockSpec((1, _BT, 2 * _H), lambda b, i, n: (b, i, 0)),
            pl.BlockSpec((1, 1, _BT), lambda b, i, n: (b * _LB + i, 0, 0)),
            pl.BlockSpec((1, 1, _BT), lambda b, i, n: (b * _LB + i, 0, 0)),
            pl.BlockSpec((_TTR, _H), lambda b, i, n: (0, 0)),
            pl.BlockSpec((_GTR, _H), lambda b, i, n: (0, 0)),
            pl.BlockSpec((1, 1, _D6), lambda b, i, n: (0, 0, 0)),
            pl.BlockSpec((1, 1, _D6), lambda b, i, n: (0, 0, 0)),
            pl.BlockSpec((_D6, _DH), lambda b, i, n: (0, 0)),
            pl.BlockSpec((1, 1, _DH), lambda b, i, n: (0, 0, 0)),
            pl.BlockSpec((_DH, _H), lambda b, i, n: (0, 0)),
            pl.BlockSpec((1, 1, _H), lambda b, i, n: (0, 0, 0)),
        ],
        out_specs=pl.BlockSpec((1, _BT, _H), lambda b, i, n: (b, i, 0)),
    )
    return pl.pallas_call(
        _mlp_body, grid_spec=grid_spec,
        out_shape=jax.ShapeDtypeStruct((_B, _L, _H), jnp.bfloat16),
    )(n_arr, xt, tid_r, gid_r, tt_pad, gt_pad, gamma, beta, w1t, b1, w2t, b2)


def _merge_body(j_ref, j2_ref, evA, evB, p_ref, s_ref, pos_ref, sep_ref,
                o_ref):
    b = pl.program_id(0)
    t = pl.program_id(1)
    j = j_ref[b * _NTB + t]
    W = 4 * _BT
    io0 = lax.broadcasted_iota(jnp.int32, (W, _BT), 0)
    io1 = lax.broadcasted_iota(jnp.int32, (W, _BT), 1)
    pid_b = jnp.broadcast_to(p_ref[0], (W, _BT))
    sl_b = jnp.broadcast_to(s_ref[0], (W, _BT))
    local = pid_b - j * _BT
    oh_ev = (io0 == local) & (sl_b == 1)
    oh_sep = (io0 == 2 * _BT) & (sl_b == 2)
    oh_pos = (io0 - 3 * _BT == io1) & (sl_b != 0)
    ohT = (oh_ev | oh_sep | oh_pos).astype(jnp.bfloat16)
    win = jnp.concatenate([evA[0], evB[0], sep_ref[0], pos_ref[...]],
                          axis=0)
    o_ref[0] = lax.dot_general(
        ohT, win, dimension_numbers=(((0,), (0,)), ((), ())),
        preferred_element_type=jnp.float32)


def _merge(ev, pidx3, sel3, j_arr, j2_arr, pos_tab, sep3):
    grid_spec = pltpu.PrefetchScalarGridSpec(
        num_scalar_prefetch=2,
        grid=(_B, _NTB),
        in_specs=[
            pl.BlockSpec((1, _BT, _H),
                         lambda b, t, j, j2: (b, j[b * _NTB + t], 0)),
            pl.BlockSpec((1, _BT, _H),
                         lambda b, t, j, j2: (b, j2[b * _NTB + t], 0)),
            pl.BlockSpec((1, 1, _BT),
                         lambda b, t, j, j2: (b * _NTB + t, 0, 0)),
            pl.BlockSpec((1, 1, _BT),
                         lambda b, t, j, j2: (b * _NTB + t, 0, 0)),
            pl.BlockSpec((_BT, _H), lambda b, t, j, j2: (t, 0)),
            pl.BlockSpec((1, _BT, _H), lambda b, t, j, j2: (0, 0, 0)),
        ],
        out_specs=pl.BlockSpec((1, _BT, _H), lambda b, t, j, j2: (b, t, 0)),
    )
    return pl.pallas_call(
        _merge_body, grid_spec=grid_spec,
        out_shape=jax.ShapeDtypeStruct((_B, _T, _H), jnp.float32),
    )(j_arr, j2_arr, ev, ev, pidx3, sel3, pos_tab, sep3)


def kernel(history_tokens, history_post_tokens, history_author_tokens,
           history_action_tokens, history_time_gap, history_group_ids,
           history_mask, token_table, time_table, group_table, pos_table,
           ln_gamma, ln_beta, W1, b1, W2, b2, sep_token):
    i32 = jnp.int32
    mask = history_mask.astype(bool)
    group = history_group_ids.astype(i32)

    mi = mask.astype(i32)
    n_arr = jnp.sum(mi, axis=1).astype(i32)
    idx = jnp.arange(_L, dtype=i32)
    packed_l = jnp.sort(jnp.where(mask, idx[None, :], _L - 1), axis=1)

    big = jnp.int32(1 << 30)
    gv = jnp.arange(9, dtype=i32)
    cnt = jnp.sum((group[:, None, :] == gv[None, :, None]) &
                  mask[:, None, :], axis=2).astype(i32)
    cum = jnp.cumsum(cnt, axis=1)
    nonempty = cnt > 0
    rev = jnp.cumsum(nonempty[:, ::-1].astype(i32), axis=1)[:, ::-1]
    sep_ex = nonempty & ((rev - nonempty.astype(i32)) > 0)
    sep_rank = jnp.cumsum(sep_ex.astype(i32), axis=1) - sep_ex.astype(i32)
    S = jnp.where(sep_ex, cum + sep_rank, big)
    tot = n_arr + jnp.sum(sep_ex, axis=1).astype(i32)

    k_t = jnp.arange(_T, dtype=i32)[None, :] - (_T - tot[:, None])
    nsep_le = jnp.sum((S[:, :, None] <= k_t[:, None, :]), axis=1)
    is_sep = jnp.any(S[:, :, None] == k_t[:, None, :], axis=1)
    sel = jnp.where(k_t >= 0, jnp.where(is_sep, 2, 1), 0)
    pidx = jnp.where(sel == 1, k_t - nsep_le, -1)
    p4 = pidx.reshape(_B, _NTB, _BT)
    w0 = jnp.min(jnp.where(p4 >= 0, p4, big), axis=2)
    j_arr = jnp.clip(jnp.where(w0 >= big, 0, w0 // _BT), 0, _LB - 1)
    j2_arr = jnp.minimum(j_arr + 1, _LB - 1)
    j_arr = j_arr.reshape(-1).astype(i32)
    j2_arr = j2_arr.reshape(-1).astype(i32)

    ids_all = jnp.stack(
        [history_tokens.astype(i32), history_post_tokens.astype(i32),
         history_author_tokens.astype(i32),
         history_action_tokens.astype(i32),
         jnp.clip(history_time_gap, 0, 128).astype(i32), group], axis=1)
    packed_all = jnp.take_along_axis(
        ids_all, jnp.broadcast_to(packed_l[:, None, :], (_B, 6, _L)),
        axis=2)
    ids4 = packed_all[:, :4].transpose(0, 2, 1).reshape(-1)
    tid_r = packed_all[:, 4].reshape(_B * _LB, 1, _BT)
    gid_r = packed_all[:, 5].reshape(_B * _LB, 1, _BT)
    bf16 = jnp.bfloat16
    tt_pad = jnp.zeros((_TTR, _H), bf16).at[:129].set(
        time_table.astype(bf16))
    gt_pad = jnp.zeros((_GTR, _H), bf16).at[:9].set(
        group_table.astype(bf16))

    u16v = lax.bitcast_convert_type(token_table.astype(bf16), jnp.uint16)
    tok_i32 = lax.bitcast_convert_type(
        u16v[:, :128].astype(jnp.uint32) |
        (u16v[:, 128:].astype(jnp.uint32) << 16), i32)
    xt4 = _sc_gather4(tok_i32, ids4).reshape(_B, _L, 2 * _H)

    def hperm(v):
        v4 = v[:1024].reshape(4, 2, 128, -1)
        return jnp.concatenate(
            [v4[:, 0].reshape(512, -1), v4[:, 1].reshape(512, -1),
             v[1024:].reshape(512, -1)], axis=0)

    gamma = hperm(ln_gamma).reshape(1, 1, _D6)
    beta = hperm(ln_beta).reshape(1, 1, _D6)
    w1t = hperm(W1.T).astype(bf16)
    w2t = W2.T.astype(bf16)
    ev = _mlp(xt4, tid_r, gid_r, tt_pad, gt_pad, n_arr, gamma, beta, w1t,
              b1.reshape(1, 1, _DH), w2t, b2.reshape(1, 1, _H))

    pidx3 = pidx.reshape(_B * _NTB, 1, _BT)
    sel3 = sel.reshape(_B * _NTB, 1, _BT)
    sep_pad = jnp.zeros((1, _BT, _H), jnp.bfloat16).at[0, 0].set(
        sep_token.astype(jnp.bfloat16))
    merged = _merge(ev, pidx3, sel3, j_arr, j2_arr,
                    pos_table.astype(jnp.bfloat16), sep_pad)
    return merged, sel != 0


# speedup vs baseline: 20.3717x; 1.0356x over previous
import functools

import jax
import jax.numpy as jnp
from jax import lax
from jax.experimental import pallas as pl
from jax.experimental.pallas import tpu as pltpu
from jax.experimental.pallas import tpu_sc as plsc

_B, _L, _T, _H = 16, 2048, 4096, 256
_NF = _B * _L
_CH = 128
_NW = 32
_BT = 256
_NTB = _T // _BT
_LB = _L // _BT
_D6 = 6 * _H
_DH = 4 * _H
_TTR = 136
_BTE = 512
_LBE = _L // _BTE
_GTR = 16


def _sc_gather4(tok_tbl, ids4):
    n4 = 4 * _NF // _NW
    ch = 128
    nb = 4
    nc = n4 // ch
    mesh = plsc.VectorSubcoreMesh(core_axis_name="c", subcore_axis_name="s")
    hw = _H // 2
    out_t = jax.ShapeDtypeStruct((4 * _NF, hw), jnp.int32)

    @functools.partial(
        pl.kernel, mesh=mesh, out_type=out_t,
        scratch_types=[pltpu.VMEM((n4,), jnp.int32),
                       pltpu.VMEM((nb, ch, hw), jnp.int32),
                       pltpu.SemaphoreType.DMA((nb,)),
                       pltpu.SemaphoreType.DMA((nb,))])
    def k(tt, i4, o4, i4_v, buf, sg, ss):
        wid = lax.axis_index("s") * 2 + lax.axis_index("c")
        base = wid * n4

        def g_start(c, par):
            pltpu.async_copy(tt.at[i4_v.at[pl.ds(c * ch, ch)]],
                             buf.at[par], sg.at[par])

        def g_wait(par):
            pltpu.make_async_copy(tt.at[i4_v.at[pl.ds(0, ch)]],
                                  buf.at[par], sg.at[par]).wait()

        def s_start(c, par):
            pltpu.async_copy(buf.at[par],
                             o4.at[pl.ds(base + c * ch, ch)],
                             ss.at[par])

        def s_wait(par):
            pltpu.make_async_copy(buf.at[0],
                                  o4.at[pl.ds(base, ch)],
                                  ss.at[par]).wait()

        pltpu.sync_copy(i4.at[pl.ds(base, n4)], i4_v)
        g_start(0, 0)
        g_start(1, 1)
        g_start(2, 2)
        g_wait(0)
        s_start(0, 0)
        g_start(3, 3)
        for k in (1, 2, 3):
            g_wait(k % nb)
            s_wait((k - 1) % nb)
            s_start(k, k % nb)
            g_start(k + 3, (k + 3) % nb)

        def body(p, carry):
            for par in (0, 1, 2, 3):
                k = 4 * p + par
                g_wait(par)
                s_wait((par - 1) % nb)
                s_start(k, par)
                g_start(k + 3, (par + 3) % nb)
            return carry
        lax.fori_loop(1, (nc - 4) // nb, body, 0)
        k = nc - 4
        g_wait(k % nb)
        s_wait((k - 1) % nb)
        s_start(k, k % nb)
        g_start(k + 3, (k + 3) % nb)
        for k in (nc - 3, nc - 2, nc - 1):
            g_wait(k % nb)
            s_wait((k - 1) % nb)
            s_start(k, k % nb)
        s_wait((nc - 1) % nb)

    return k(tok_tbl, ids4)


def _mlp_body(n_ref, x0, tid_ref, gid_ref, tt_ref, gt_ref, g_ref, be_ref,
              w1_ref, b1_ref, w2_ref, b2_ref, o_ref):
    b = pl.program_id(0)
    i = pl.program_id(1)
    nb = n_ref[b]

    @pl.when(i * _BTE < nb)
    def _compute():
        wu = lax.bitcast_convert_type(x0[0], jnp.uint32)
        lo = lax.bitcast_convert_type(wu.astype(jnp.uint16), jnp.bfloat16)
        hi = lax.bitcast_convert_type((wu >> 16).astype(jnp.uint16),
                                      jnp.bfloat16)
        xtok = jnp.concatenate([lo, hi], axis=-1).astype(jnp.float32)
        tn = (((0,), (0,)), ((), ()))
        iot = lax.broadcasted_iota(jnp.int32, (_TTR, _BTE), 0)
        oht = (iot == jnp.broadcast_to(tid_ref[0], (_TTR, _BTE))).astype(
            jnp.bfloat16)
        th = lax.dot_general(oht, tt_ref[...], dimension_numbers=tn,
                             preferred_element_type=jnp.float32)
        iog = lax.broadcasted_iota(jnp.int32, (_GTR, _BTE), 0)
        ohg = (iog == jnp.broadcast_to(gid_ref[0], (_GTR, _BTE))).astype(
            jnp.bfloat16)
        gh = lax.dot_general(ohg, gt_ref[...], dimension_numbers=tn,
                             preferred_element_type=jnp.float32)
        x = jnp.concatenate([xtok, th, gh], axis=-1)
        mu = jnp.mean(x, axis=1, keepdims=True)
        var = jnp.mean(x * x, axis=1, keepdims=True) - mu * mu
        xn = (x - mu) * lax.rsqrt(var + 1e-5)
        xn = xn * g_ref[0] + be_ref[0]
        h = jnp.dot(xn.astype(jnp.bfloat16), w1_ref[...],
                    preferred_element_type=jnp.float32) + b1_ref[0]
        a = h * jax.nn.sigmoid(h)
        o = jnp.dot(a.astype(jnp.bfloat16), w2_ref[...],
                    preferred_element_type=jnp.float32) + b2_ref[0]
        o_ref[0] = o.astype(jnp.bfloat16)

    @pl.when(i * _BTE >= nb)
    def _zero():
        o_ref[...] = jnp.zeros_like(o_ref)


def _mlp(xt, tid_r, gid_r, tt_pad, gt_pad, n_arr, gamma, beta,
         w1t, b1, w2t, b2):
    grid_spec = pltpu.PrefetchScalarGridSpec(
        num_scalar_prefetch=1,
        grid=(_B, _LBE),
        in_specs=[
            pl.BlockSpec((1, _BTE, 2 * _H), lambda b, i, n: (b, i, 0)),
            pl.BlockSpec((1, 1, _BTE), lambda b, i, n: (b * _LBE + i, 0, 0)),
            pl.BlockSpec((1, 1, _BTE), lambda b, i, n: (b * _LBE + i, 0, 0)),
            pl.BlockSpec((_TTR, _H), lambda b, i, n: (0, 0)),
            pl.BlockSpec((_GTR, _H), lambda b, i, n: (0, 0)),
            pl.BlockSpec((1, 1, _D6), lambda b, i, n: (0, 0, 0)),
            pl.BlockSpec((1, 1, _D6), lambda b, i, n: (0, 0, 0)),
            pl.BlockSpec((_D6, _DH), lambda b, i, n: (0, 0)),
            pl.BlockSpec((1, 1, _DH), lambda b, i, n: (0, 0, 0)),
            pl.BlockSpec((_DH, _H), lambda b, i, n: (0, 0)),
            pl.BlockSpec((1, 1, _H), lambda b, i, n: (0, 0, 0)),
        ],
        out_specs=pl.BlockSpec((1, _BTE, _H), lambda b, i, n: (b, i, 0)),
    )
    return pl.pallas_call(
        _mlp_body, grid_spec=grid_spec,
        out_shape=jax.ShapeDtypeStruct((_B, _L, _H), jnp.bfloat16),
    )(n_arr, xt, tid_r, gid_r, tt_pad, gt_pad, gamma, beta, w1t, b1, w2t, b2)


def _merge_body(j_ref, j2_ref, evA, evB, p_ref, s_ref, pos_ref, sep_ref,
                o_ref):
    b = pl.program_id(0)
    t = pl.program_id(1)
    j = j_ref[b * _NTB + t]
    W = 4 * _BT
    io0 = lax.broadcasted_iota(jnp.int32, (W, _BT), 0)
    io1 = lax.broadcasted_iota(jnp.int32, (W, _BT), 1)
    pid_b = jnp.broadcast_to(p_ref[0], (W, _BT))
    sl_b = jnp.broadcast_to(s_ref[0], (W, _BT))
    local = pid_b - j * _BT
    oh_ev = (io0 == local) & (sl_b == 1)
    oh_sep = (io0 == 2 * _BT) & (sl_b == 2)
    oh_pos = (io0 - 3 * _BT == io1) & (sl_b != 0)
    ohT = (oh_ev | oh_sep | oh_pos).astype(jnp.bfloat16)
    win = jnp.concatenate([evA[0], evB[0], sep_ref[0], pos_ref[...]],
                          axis=0)
    o_ref[0] = lax.dot_general(
        ohT, win, dimension_numbers=(((0,), (0,)), ((), ())),
        preferred_element_type=jnp.float32)


def _merge(ev, pidx3, sel3, j_arr, j2_arr, pos_tab, sep3):
    grid_spec = pltpu.PrefetchScalarGridSpec(
        num_scalar_prefetch=2,
        grid=(_B, _NTB),
        in_specs=[
            pl.BlockSpec((1, _BT, _H),
                         lambda b, t, j, j2: (b, j[b * _NTB + t], 0)),
            pl.BlockSpec((1, _BT, _H),
                         lambda b, t, j, j2: (b, j2[b * _NTB + t], 0)),
            pl.BlockSpec((1, 1, _BT),
                         lambda b, t, j, j2: (b * _NTB + t, 0, 0)),
            pl.BlockSpec((1, 1, _BT),
                         lambda b, t, j, j2: (b * _NTB + t, 0, 0)),
            pl.BlockSpec((_BT, _H), lambda b, t, j, j2: (t, 0)),
            pl.BlockSpec((1, _BT, _H), lambda b, t, j, j2: (0, 0, 0)),
        ],
        out_specs=pl.BlockSpec((1, _BT, _H), lambda b, t, j, j2: (b, t, 0)),
    )
    return pl.pallas_call(
        _merge_body, grid_spec=grid_spec,
        out_shape=jax.ShapeDtypeStruct((_B, _T, _H), jnp.float32),
    )(j_arr, j2_arr, ev, ev, pidx3, sel3, pos_tab, sep3)


def kernel(history_tokens, history_post_tokens, history_author_tokens,
           history_action_tokens, history_time_gap, history_group_ids,
           history_mask, token_table, time_table, group_table, pos_table,
           ln_gamma, ln_beta, W1, b1, W2, b2, sep_token):
    i32 = jnp.int32
    mask = history_mask.astype(bool)
    group = history_group_ids.astype(i32)

    mi = mask.astype(i32)
    n_arr = jnp.sum(mi, axis=1).astype(i32)
    idx = jnp.arange(_L, dtype=i32)
    packed_l = jnp.sort(jnp.where(mask, idx[None, :], _L - 1), axis=1)

    big = jnp.int32(1 << 30)
    gv = jnp.arange(9, dtype=i32)
    cnt = jnp.sum((group[:, None, :] == gv[None, :, None]) &
                  mask[:, None, :], axis=2).astype(i32)
    cum = jnp.cumsum(cnt, axis=1)
    nonempty = cnt > 0
    rev = jnp.cumsum(nonempty[:, ::-1].astype(i32), axis=1)[:, ::-1]
    sep_ex = nonempty & ((rev - nonempty.astype(i32)) > 0)
    sep_rank = jnp.cumsum(sep_ex.astype(i32), axis=1) - sep_ex.astype(i32)
    S = jnp.where(sep_ex, cum + sep_rank, big)
    tot = n_arr + jnp.sum(sep_ex, axis=1).astype(i32)

    k_t = jnp.arange(_T, dtype=i32)[None, :] - (_T - tot[:, None])
    nsep_le = jnp.sum((S[:, :, None] <= k_t[:, None, :]), axis=1)
    is_sep = jnp.any(S[:, :, None] == k_t[:, None, :], axis=1)
    sel = jnp.where(k_t >= 0, jnp.where(is_sep, 2, 1), 0)
    pidx = jnp.where(sel == 1, k_t - nsep_le, -1)
    p4 = pidx.reshape(_B, _NTB, _BT)
    w0 = jnp.min(jnp.where(p4 >= 0, p4, big), axis=2)
    j_arr = jnp.clip(jnp.where(w0 >= big, 0, w0 // _BT), 0, _LB - 1)
    j2_arr = jnp.minimum(j_arr + 1, _LB - 1)
    j_arr = j_arr.reshape(-1).astype(i32)
    j2_arr = j2_arr.reshape(-1).astype(i32)

    ids_all = jnp.stack(
        [history_tokens.astype(i32), history_post_tokens.astype(i32),
         history_author_tokens.astype(i32),
         history_action_tokens.astype(i32),
         jnp.clip(history_time_gap, 0, 128).astype(i32), group], axis=1)
    packed_all = jnp.take_along_axis(
        ids_all, jnp.broadcast_to(packed_l[:, None, :], (_B, 6, _L)),
        axis=2)
    ids4 = packed_all[:, :4].transpose(0, 2, 1).reshape(-1)
    tid_r = packed_all[:, 4].reshape(_B * _LBE, 1, _BTE)
    gid_r = packed_all[:, 5].reshape(_B * _LBE, 1, _BTE)
    bf16 = jnp.bfloat16
    tt_pad = jnp.zeros((_TTR, _H), bf16).at[:129].set(
        time_table.astype(bf16))
    gt_pad = jnp.zeros((_GTR, _H), bf16).at[:9].set(
        group_table.astype(bf16))

    u16v = lax.bitcast_convert_type(token_table.astype(bf16), jnp.uint16)
    tok_i32 = lax.bitcast_convert_type(
        u16v[:, :128].astype(jnp.uint32) |
        (u16v[:, 128:].astype(jnp.uint32) << 16), i32)
    xt4 = _sc_gather4(tok_i32, ids4).reshape(_B, _L, 2 * _H)

    def hperm(v):
        v4 = v[:1024].reshape(4, 2, 128, -1)
        return jnp.concatenate(
            [v4[:, 0].reshape(512, -1), v4[:, 1].reshape(512, -1),
             v[1024:].reshape(512, -1)], axis=0)

    gamma = hperm(ln_gamma).reshape(1, 1, _D6)
    beta = hperm(ln_beta).reshape(1, 1, _D6)
    w1t = hperm(W1.T).astype(bf16)
    w2t = W2.T.astype(bf16)
    ev = _mlp(xt4, tid_r, gid_r, tt_pad, gt_pad, n_arr, gamma, beta, w1t,
              b1.reshape(1, 1, _DH), w2t, b2.reshape(1, 1, _H))

    pidx3 = pidx.reshape(_B * _NTB, 1, _BT)
    sel3 = sel.reshape(_B * _NTB, 1, _BT)
    sep_pad = jnp.zeros((1, _BT, _H), jnp.bfloat16).at[0, 0].set(
        sep_token.astype(jnp.bfloat16))
    merged = _merge(ev, pidx3, sel3, j_arr, j2_arr,
                    pos_table.astype(jnp.bfloat16), sep_pad)
    return merged, sel != 0
